# baseline (speedup 1.0000x reference)
import jax
import jax.numpy as jnp
from jax.experimental import pallas as pl
from jax.experimental.pallas import tpu as pltpu

_NQ = 4


def _cdfg_kernel(idx_ref, xs_ref, m_ref, win_ref, bin_ref,
                 w1_ref, b1_ref, w2_ref, b2_ref, a_hbm, out_ref,
                 a_scr, sems):
    b = pl.program_id(0)
    nb = pl.num_programs(0)
    n = a_scr.shape[1]
    nq = n // _NQ
    slot = jax.lax.rem(b, 2)
    nslot = jax.lax.rem(b + 1, 2)

    def quarter_copy(dst_slot, g, q):
        return pltpu.make_async_copy(
            a_hbm.at[g, pl.ds(q * nq, nq), :],
            a_scr.at[dst_slot, pl.ds(q * nq, nq), :],
            sems.at[dst_slot, q])

    g_here = idx_ref[b, 0]
    g_next = idx_ref[jnp.minimum(b + 1, nb - 1), 0]

    @pl.when(b == 0)
    def _warmup():
        for q in range(_NQ):
            quarter_copy(0, g_here, q).start()

    @pl.when(b + 1 < nb)
    def _prefetch_next():
        for q in range(_NQ):
            quarter_copy(nslot, g_next, q).start()

    xs = xs_ref[0]
    m = m_ref[b][None, :]
    x0 = jnp.maximum(
        jnp.dot(xs, win_ref[...], preferred_element_type=jnp.float32)
        + bin_ref[...], 0.0)
    y1 = jnp.dot(x0, w1_ref[...], preferred_element_type=jnp.float32)

    x1_parts = []
    for q in range(_NQ):
        quarter_copy(slot, g_here, q).wait()
        aq = a_scr[slot, pl.ds(q * nq, nq), :]
        x1_parts.append(jnp.maximum(
            jnp.dot(aq, y1, preferred_element_type=jnp.float32)
            + b1_ref[...], 0.0))
    x1 = jnp.concatenate(x1_parts, axis=0)
    y2 = jnp.dot(x1, w2_ref[...], preferred_element_type=jnp.float32)
    x2 = jnp.tanh(
        jnp.dot(a_scr[slot], y2, preferred_element_type=jnp.float32)
        + b2_ref[...])
    x = x2 + x0
    num = jnp.dot(m, x, preferred_element_type=jnp.float32)
    den = jnp.sum(m)
    out_ref[b, :] = (num / den)[0]


def kernel(graph, coverpoint, coverpoint_mask, batch_xs, batch_as,
           W_in, b_in, W1, b1, W2, b2):
    B = graph.shape[0]
    _, N, F = batch_xs.shape
    H = W1.shape[1]

    grid_spec = pltpu.PrefetchScalarGridSpec(
        num_scalar_prefetch=1,
        grid=(B,),
        in_specs=[
            pl.BlockSpec((1, N, F), lambda b, i: (i[b, 0], 0, 0)),
            pl.BlockSpec((B, N), lambda b, i: (0, 0)),
            pl.BlockSpec((F, H), lambda b, i: (0, 0)),
            pl.BlockSpec((1, H), lambda b, i: (0, 0)),
            pl.BlockSpec((H, H), lambda b, i: (0, 0)),
            pl.BlockSpec((1, H), lambda b, i: (0, 0)),
            pl.BlockSpec((H, H), lambda b, i: (0, 0)),
            pl.BlockSpec((1, H), lambda b, i: (0, 0)),
            pl.BlockSpec(memory_space=pl.MemorySpace.ANY),
        ],
        out_specs=pl.BlockSpec((B, H), lambda b, i: (0, 0)),
        scratch_shapes=[
            pltpu.VMEM((2, N, N), jnp.float32),
            pltpu.SemaphoreType.DMA((2, _NQ)),
        ],
    )
    return pl.pallas_call(
        _cdfg_kernel,
        grid_spec=grid_spec,
        out_shape=jax.ShapeDtypeStruct((B, H), jnp.float32),
        compiler_params=pltpu.CompilerParams(
            vmem_limit_bytes=100 * 1024 * 1024),
    )(graph, batch_xs, coverpoint_mask.astype(jnp.float32),
      W_in, b_in.reshape(1, -1), W1, b1.reshape(1, -1), W2, b2.reshape(1, -1),
      batch_as)


# speedup vs baseline: 1.1672x; 1.1672x over previous
import jax
import jax.numpy as jnp
from jax.experimental import pallas as pl
from jax.experimental.pallas import tpu as pltpu


def _cdfg_kernel(idx_ref, xs_ref, a0_ref, a1_ref, m_ref,
                 win_ref, bin_ref, w1_ref, b1_ref, w2_ref, b2_ref, out_ref):
    b = pl.program_id(0)
    xs = xs_ref[0]
    m = m_ref[0].astype(jnp.float32)

    def conv(y):
        return jnp.concatenate(
            [jnp.dot(p[0], y, preferred_element_type=jnp.float32)
             for p in (a0_ref, a1_ref)], axis=0)

    x0 = jnp.maximum(
        jnp.dot(xs, win_ref[...], preferred_element_type=jnp.float32)
        + bin_ref[...], 0.0)
    y1 = jnp.dot(x0, w1_ref[...], preferred_element_type=jnp.float32)
    x1 = jnp.maximum(conv(y1) + b1_ref[...], 0.0)
    y2 = jnp.dot(x1, w2_ref[...], preferred_element_type=jnp.float32)
    x2 = jnp.tanh(conv(y2) + b2_ref[...])
    x = x2 + x0
    num = jnp.dot(m, x, preferred_element_type=jnp.float32)
    den = jnp.sum(m)
    out_ref[b, :] = (num / den)[0]


def kernel(graph, coverpoint, coverpoint_mask, batch_xs, batch_as,
           W_in, b_in, W1, b1, W2, b2):
    B = graph.shape[0]
    _, N, F = batch_xs.shape
    H = W1.shape[1]
    NH = N // 2

    grid_spec = pltpu.PrefetchScalarGridSpec(
        num_scalar_prefetch=1,
        grid=(B,),
        in_specs=[
            pl.BlockSpec((1, N, F), lambda b, i: (i[b, 0], 0, 0)),
            pl.BlockSpec((1, NH, N), lambda b, i: (i[b, 0], 0, 0)),
            pl.BlockSpec((1, NH, N), lambda b, i: (i[b, 0], 1, 0)),
            pl.BlockSpec((1, 1, N), lambda b, i: (b, 0, 0)),
            pl.BlockSpec((F, H), lambda b, i: (0, 0)),
            pl.BlockSpec((1, H), lambda b, i: (0, 0)),
            pl.BlockSpec((H, H), lambda b, i: (0, 0)),
            pl.BlockSpec((1, H), lambda b, i: (0, 0)),
            pl.BlockSpec((H, H), lambda b, i: (0, 0)),
            pl.BlockSpec((1, H), lambda b, i: (0, 0)),
        ],
        out_specs=pl.BlockSpec((B, H), lambda b, i: (0, 0)),
    )
    return pl.pallas_call(
        _cdfg_kernel,
        grid_spec=grid_spec,
        out_shape=jax.ShapeDtypeStruct((B, H), jnp.float32),
        compiler_params=pltpu.CompilerParams(
            vmem_limit_bytes=100 * 1024 * 1024),
    )(graph, batch_xs, batch_as, batch_as,
      coverpoint_mask.view(jnp.int8).reshape(B, 1, N),
      W_in, b_in.reshape(1, -1), W1, b1.reshape(1, -1), W2, b2.reshape(1, -1))


# speedup vs baseline: 1.1708x; 1.0031x over previous
import jax
import jax.numpy as jnp
from jax.experimental import pallas as pl
from jax.experimental.pallas import tpu as pltpu


def _cdfg_kernel(idx_ref, xs_ref, a0_ref, a1_ref, m_ref,
                 win_ref, bin_ref, w1_ref, b1_ref, w2_ref, b2_ref, out_ref):
    b = pl.program_id(0)
    xs = xs_ref[0]
    m = m_ref[0].astype(jnp.float32)

    def conv(y):
        return jnp.concatenate(
            [jnp.dot(p[0], y, preferred_element_type=jnp.float32)
             for p in (a0_ref, a1_ref)], axis=0)

    x0 = jnp.maximum(
        jnp.dot(xs, win_ref[...], preferred_element_type=jnp.float32)
        + bin_ref[...], 0.0)
    y1 = jnp.dot(x0, w1_ref[...], preferred_element_type=jnp.float32)
    x1 = jnp.maximum(conv(y1) + b1_ref[...], 0.0)
    y2 = jnp.dot(x1, w2_ref[...], preferred_element_type=jnp.float32)
    x2 = jnp.tanh(conv(y2) + b2_ref[...])
    x = x2 + x0
    num = jnp.dot(m, x, preferred_element_type=jnp.float32)
    den = jnp.sum(m)
    out_ref[b, :] = (num / den)[0]


def kernel(graph, coverpoint, coverpoint_mask, batch_xs, batch_as,
           W_in, b_in, W1, b1, W2, b2):
    B = graph.shape[0]
    _, N, F = batch_xs.shape
    H = W1.shape[1]
    NH = N // 2

    grid_spec = pltpu.PrefetchScalarGridSpec(
        num_scalar_prefetch=1,
        grid=(B,),
        in_specs=[
            pl.BlockSpec((1, N, F), lambda b, i: (i[b, 0], 0, 0)),
            pl.BlockSpec((1, NH, N), lambda b, i: (i[b, 0], 0, 0)),
            pl.BlockSpec((1, NH, N), lambda b, i: (i[b, 0], 1, 0)),
            pl.BlockSpec((1, 1, N), lambda b, i: (b, 0, 0)),
            pl.BlockSpec((F, H), lambda b, i: (0, 0)),
            pl.BlockSpec((1, H), lambda b, i: (0, 0)),
            pl.BlockSpec((H, H), lambda b, i: (0, 0)),
            pl.BlockSpec((1, H), lambda b, i: (0, 0)),
            pl.BlockSpec((H, H), lambda b, i: (0, 0)),
            pl.BlockSpec((1, H), lambda b, i: (0, 0)),
        ],
        out_specs=pl.BlockSpec((B, H), lambda b, i: (0, 0)),
    )
    return pl.pallas_call(
        _cdfg_kernel,
        grid_spec=grid_spec,
        out_shape=jax.ShapeDtypeStruct((B, H), jnp.float32),
        compiler_params=pltpu.CompilerParams(
            vmem_limit_bytes=100 * 1024 * 1024),
    )(graph, batch_xs, batch_as, batch_as,
      coverpoint_mask.reshape(B, 1, N),
      W_in, b_in.reshape(1, -1), W1, b1.reshape(1, -1), W2, b2.reshape(1, -1))


# speedup vs baseline: 1.1737x; 1.0025x over previous
import jax
import jax.numpy as jnp
from jax.experimental import pallas as pl
from jax.experimental.pallas import tpu as pltpu


def _cdfg_kernel(idx_ref, xs_ref, a0_ref, a1_ref, m_ref,
                 win_ref, bin_ref, w1_ref, b1_ref, w2_ref, b2_ref, out_ref):
    b = pl.program_id(0)
    xs = xs_ref[0]
    m = m_ref[0].astype(jnp.float32)

    def conv(y):
        return jnp.concatenate(
            [jnp.dot(p[0], y, preferred_element_type=jnp.float32,
                     precision=jax.lax.Precision.DEFAULT)
             for p in (a0_ref, a1_ref)], axis=0)

    x0 = jnp.maximum(
        jnp.dot(xs, win_ref[...], preferred_element_type=jnp.float32)
        + bin_ref[...], 0.0)
    y1 = jnp.dot(x0, w1_ref[...], preferred_element_type=jnp.float32)
    x1 = jnp.maximum(conv(y1) + b1_ref[...], 0.0)
    y2 = jnp.dot(x1, w2_ref[...], preferred_element_type=jnp.float32)
    x2 = jnp.tanh(conv(y2) + b2_ref[...])
    x = x2 + x0
    num = jnp.dot(m, x, preferred_element_type=jnp.float32)
    den = jnp.sum(m)
    out_ref[b, :] = (num / den)[0]


def kernel(graph, coverpoint, coverpoint_mask, batch_xs, batch_as,
           W_in, b_in, W1, b1, W2, b2):
    B = graph.shape[0]
    _, N, F = batch_xs.shape
    H = W1.shape[1]
    NH = N // 2

    grid_spec = pltpu.PrefetchScalarGridSpec(
        num_scalar_prefetch=1,
        grid=(B,),
        in_specs=[
            pl.BlockSpec((1, N, F), lambda b, i: (i[b, 0], 0, 0)),
            pl.BlockSpec((1, NH, N), lambda b, i: (i[b, 0], 0, 0)),
            pl.BlockSpec((1, NH, N), lambda b, i: (i[b, 0], 1, 0)),
            pl.BlockSpec((1, 1, N), lambda b, i: (b, 0, 0)),
            pl.BlockSpec((F, H), lambda b, i: (0, 0)),
            pl.BlockSpec((1, H), lambda b, i: (0, 0)),
            pl.BlockSpec((H, H), lambda b, i: (0, 0)),
            pl.BlockSpec((1, H), lambda b, i: (0, 0)),
            pl.BlockSpec((H, H), lambda b, i: (0, 0)),
            pl.BlockSpec((1, H), lambda b, i: (0, 0)),
        ],
        out_specs=pl.BlockSpec((B, H), lambda b, i: (0, 0)),
    )
    return pl.pallas_call(
        _cdfg_kernel,
        grid_spec=grid_spec,
        out_shape=jax.ShapeDtypeStruct((B, H), jnp.float32),
        compiler_params=pltpu.CompilerParams(
            vmem_limit_bytes=100 * 1024 * 1024),
    )(graph, batch_xs, batch_as, batch_as,
      coverpoint_mask.reshape(B, 1, N),
      W_in, b_in.reshape(1, -1), W1, b1.reshape(1, -1), W2, b2.reshape(1, -1))
